# baseline (speedup 1.0000x reference)
import functools

import jax
import jax.numpy as jnp
from jax import lax
from jax.experimental import pallas as pl
from jax.experimental.pallas import tpu as pltpu
from jax.experimental.pallas import tpu_sc as plsc

_VOCAB = 1000
_VPAD = 1024
_B = 1024
_L = 50
_LPAD = 56
_N = _B * _L
_NC = 2
_NS = 16
_NW = _NC * _NS
_BPW = _B // _NW
_PER_W = _N // _NW


def _lse_body(emb_ref, out_ref):
    x = emb_ref[...]
    m = jnp.max(x, axis=1, keepdims=True)
    s = jnp.sum(jnp.exp(x - m), axis=1, keepdims=True)
    out_ref[...] = jnp.log(s) + m


def _compute_lse(embedding):
    out = pl.pallas_call(
        _lse_body,
        out_shape=jax.ShapeDtypeStruct((_VOCAB, 1), jnp.float32),
    )(embedding)
    return out.reshape(_VOCAB)


_sc_mesh = plsc.VectorSubcoreMesh(core_axis_name="c", subcore_axis_name="s")


@functools.partial(
    pl.kernel,
    out_type=jax.ShapeDtypeStruct((_B, _LPAD, _VPAD), jnp.float32),
    mesh=_sc_mesh,
    compiler_params=pltpu.CompilerParams(use_tc_tiling_on_sc=True),
    scratch_types=[
        pltpu.VMEM((_BPW, _LPAD), jnp.int32),
        pltpu.VMEM((_LPAD, _VPAD), jnp.float32),
        pltpu.VMEM((_LPAD, _VPAD), jnp.float32),
        pltpu.SemaphoreType.DMA,
        pltpu.SemaphoreType.DMA,
        pltpu.SemaphoreType.DMA,
        pltpu.SemaphoreType.DMA,
    ],
)
def _sc_rows(emb_pad_hbm, idx56_hbm, out_hbm,
             idx_v, rows0, rows1, sem_g0, sem_g1, sem_w0, sem_w1):
    wid = lax.axis_index("s") * _NC + lax.axis_index("c")
    wb = wid * _BPW
    pltpu.sync_copy(idx56_hbm.at[pl.ds(wb, _BPW)], idx_v)

    def issue_gather(c, buf, sem):
        pltpu.async_copy(emb_pad_hbm.at[idx_v.at[c]], buf, sem)

    def wait_gather(buf, sem):
        pltpu.make_async_copy(emb_pad_hbm.at[idx_v.at[0]], buf, sem).wait()

    def issue_write(c, buf, sem):
        pltpu.async_copy(buf, out_hbm.at[wb + c], sem)

    def wait_write(buf, sem):
        pltpu.make_async_copy(buf, out_hbm.at[wb], sem).wait()

    issue_gather(0, rows0, sem_g0)
    issue_gather(1, rows1, sem_g1)

    def step(i, carry):
        c0 = i * 2
        wait_gather(rows0, sem_g0)
        issue_write(c0, rows0, sem_w0)
        wait_gather(rows1, sem_g1)
        issue_write(c0 + 1, rows1, sem_w1)
        wait_write(rows0, sem_w0)
        issue_gather(c0 + 2, rows0, sem_g0)
        wait_write(rows1, sem_w1)
        issue_gather(c0 + 3, rows1, sem_g1)
        return carry

    lax.fori_loop(0, _BPW // 2 - 1, step, 0)

    wait_gather(rows0, sem_g0)
    issue_write(_BPW - 2, rows0, sem_w0)
    wait_gather(rows1, sem_g1)
    issue_write(_BPW - 1, rows1, sem_w1)
    wait_write(rows0, sem_w0)
    wait_write(rows1, sem_w1)


_SEG_C = 64
_NSEG_CHUNK = _PER_W // _SEG_C


@functools.partial(
    pl.kernel,
    out_type=jax.ShapeDtypeStruct((_N,), jnp.float32),
    mesh=_sc_mesh,
    compiler_params=pltpu.CompilerParams(
        use_tc_tiling_on_sc=False, needs_layout_passes=False),
    scratch_types=[
        pltpu.VMEM((_PER_W,), jnp.int32),
        pltpu.VMEM((_PER_W,), jnp.int32),
        pltpu.VMEM((_PER_W,), jnp.int32),
        pltpu.VMEM((_SEG_C, 128), jnp.float32),
        pltpu.VMEM((_VOCAB,), jnp.float32),
        pltpu.VMEM((_PER_W,), jnp.float32),
        pltpu.SemaphoreType.DMA,
    ],
)
def _sc_loss(emb_r8_hbm, idx_hbm, tgt_hbm, lse_hbm, loss_hbm,
             idx_all, tgt_all, seg_all, seg_v, lse_v, loss_all, sem):
    wid = lax.axis_index("s") * _NC + lax.axis_index("c")
    base_w = wid * _PER_W
    pltpu.sync_copy(idx_hbm.at[pl.ds(base_w, _PER_W)], idx_all)
    pltpu.sync_copy(tgt_hbm.at[pl.ds(base_w, _PER_W)], tgt_all)
    pltpu.sync_copy(lse_hbm, lse_v)

    for k in range(_PER_W // 16):
        sl = pl.ds(k * 16, 16)
        seg_all[sl] = idx_all[sl] * 8 + jnp.right_shift(tgt_all[sl], 7)

    def chunk(m, carry):
        off = pl.multiple_of(m * _SEG_C, _SEG_C)
        pltpu.async_copy(emb_r8_hbm.at[seg_all.at[pl.ds(off, _SEG_C)]],
                         seg_v, sem).wait()
        for j in range(_SEG_C // 16):
            sl = pl.ds(off + j * 16, 16)
            idx16 = idx_all[sl]
            lane16 = jnp.bitwise_and(tgt_all[sl], 127)
            row16 = lax.iota(jnp.int32, 16) + (j * 16)
            tl16 = plsc.load_gather(seg_v, [row16, lane16])
            lse16 = plsc.load_gather(lse_v, [idx16])
            loss_all[sl] = lse16 - tl16
        return carry

    lax.fori_loop(0, _NSEG_CHUNK, chunk, 0)
    pltpu.sync_copy(loss_all, loss_hbm.at[pl.ds(base_w, _PER_W)])


def kernel(indices, targets, embedding):
    emb_pad = jnp.pad(embedding, ((0, 0), (0, _VPAD - _VOCAB)))
    emb_r8 = emb_pad.reshape(_VOCAB * 8, 128)
    idx56 = jnp.pad(indices, ((0, 0), (0, _LPAD - _L)))
    idx_flat = indices.reshape(_N)
    tgt_flat = targets.reshape(_N)
    lse = _compute_lse(embedding)
    out_p = _sc_rows(emb_pad, idx56)
    loss = _sc_loss(emb_r8, idx_flat, tgt_flat, lse)
    return out_p[:, :_L, :_VOCAB], loss


# speedup vs baseline: 1.0067x; 1.0067x over previous
import functools

import jax
import jax.numpy as jnp
from jax import lax
from jax.experimental import pallas as pl
from jax.experimental.pallas import tpu as pltpu
from jax.experimental.pallas import tpu_sc as plsc

_VOCAB = 1000
_VPAD = 1024
_B = 1024
_L = 50
_LPAD = 56
_LP64 = 64
_N = _B * _L
_NC = 2
_NS = 16
_NW = _NC * _NS
_BPW = _B // _NW
_PER_W = _N // _NW
_NSEG = 8 * _LPAD
_GCH = 112


def _lse_body(emb_ref, out_ref):
    x = emb_ref[...]
    m = jnp.max(x, axis=1, keepdims=True)
    s = jnp.sum(jnp.exp(x - m), axis=1, keepdims=True)
    out_ref[...] = jnp.log(s) + m


def _compute_lse(embedding):
    out = pl.pallas_call(
        _lse_body,
        out_shape=jax.ShapeDtypeStruct((_VOCAB, 1), jnp.float32),
    )(embedding)
    return out.reshape(_VOCAB)


_sc_mesh = plsc.VectorSubcoreMesh(core_axis_name="c", subcore_axis_name="s")


@functools.partial(
    pl.kernel,
    out_type=jax.ShapeDtypeStruct((_B, _LPAD, _VPAD), jnp.float32),
    mesh=_sc_mesh,
    compiler_params=pltpu.CompilerParams(
        use_tc_tiling_on_sc=True, needs_layout_passes=False),
    scratch_types=[
        pltpu.VMEM((_BPW * _LP64,), jnp.int32),
        pltpu.VMEM((_NSEG + 16,), jnp.int32),
        pltpu.VMEM((_NSEG + 16,), jnp.int32),
        pltpu.VMEM((_NSEG, 128), jnp.float32),
        pltpu.VMEM((_NSEG, 128), jnp.float32),
        pltpu.SemaphoreType.DMA,
        pltpu.SemaphoreType.DMA,
        pltpu.SemaphoreType.DMA,
        pltpu.SemaphoreType.DMA,
    ],
)
def _sc_rows(emb_r8_hbm, idx64_hbm, out_hbm,
             idx_v, list0, list1, buf0, buf1,
             sem_g0, sem_g1, sem_w0, sem_w1):
    wid = lax.axis_index("s") * _NC + lax.axis_index("c")
    wb = wid * _BPW
    pltpu.sync_copy(idx64_hbm.at[pl.ds(wb * _LP64, _BPW * _LP64)], idx_v)

    def build_list(c, list_v):
        base = pl.multiple_of(c * _LP64, _LP64)
        segs = []
        for j in range(4):
            l16 = lax.iota(jnp.int32, 16) + (j * 16)
            segs.append(plsc.load_gather(idx_v, [base + l16]) * 8)
        for t in range(8):
            for j in range(4):
                list_v[pl.ds(t * _LPAD + j * 16, 16)] = segs[j] + t

    def issue_gathers(list_v, buf, sem):
        for k in range(_NSEG // _GCH):
            pltpu.async_copy(
                emb_r8_hbm.at[list_v.at[pl.ds(k * _GCH, _GCH)]],
                buf.at[pl.ds(k * _GCH, _GCH)], sem)

    def wait_gathers(list_v, buf, sem):
        for k in range(_NSEG // _GCH):
            pltpu.make_async_copy(
                emb_r8_hbm.at[list_v.at[pl.ds(k * _GCH, _GCH)]],
                buf.at[pl.ds(k * _GCH, _GCH)], sem).wait()

    def issue_writes(c, buf, sem):
        for t in range(8):
            pltpu.async_copy(
                buf.at[pl.ds(t * _LPAD, _LPAD)],
                out_hbm.at[wb + c].at[pl.ds(0, _LPAD),
                                      pl.ds(t * 128, 128)], sem)

    def wait_writes(c, buf, sem):
        for t in range(8):
            pltpu.make_async_copy(
                buf.at[pl.ds(t * _LPAD, _LPAD)],
                out_hbm.at[wb + c].at[pl.ds(0, _LPAD),
                                      pl.ds(t * 128, 128)], sem).wait()

    build_list(0, list0)
    issue_gathers(list0, buf0, sem_g0)
    build_list(1, list1)
    issue_gathers(list1, buf1, sem_g1)

    def step(i, carry):
        c0 = i * 2
        wait_gathers(list0, buf0, sem_g0)
        issue_writes(c0, buf0, sem_w0)
        wait_gathers(list1, buf1, sem_g1)
        issue_writes(c0 + 1, buf1, sem_w1)
        wait_writes(c0, buf0, sem_w0)
        build_list(c0 + 2, list0)
        issue_gathers(list0, buf0, sem_g0)
        wait_writes(c0 + 1, buf1, sem_w1)
        build_list(c0 + 3, list1)
        issue_gathers(list1, buf1, sem_g1)
        return carry

    lax.fori_loop(0, _BPW // 2 - 1, step, 0)

    wait_gathers(list0, buf0, sem_g0)
    issue_writes(_BPW - 2, buf0, sem_w0)
    wait_gathers(list1, buf1, sem_g1)
    issue_writes(_BPW - 1, buf1, sem_w1)
    wait_writes(_BPW - 2, buf0, sem_w0)
    wait_writes(_BPW - 1, buf1, sem_w1)


_SEG_C = 64
_NSEG_CHUNK = _PER_W // _SEG_C


@functools.partial(
    pl.kernel,
    out_type=jax.ShapeDtypeStruct((_N,), jnp.float32),
    mesh=_sc_mesh,
    compiler_params=pltpu.CompilerParams(
        use_tc_tiling_on_sc=False, needs_layout_passes=False),
    scratch_types=[
        pltpu.VMEM((_PER_W,), jnp.int32),
        pltpu.VMEM((_PER_W,), jnp.int32),
        pltpu.VMEM((_PER_W,), jnp.int32),
        pltpu.VMEM((_SEG_C, 128), jnp.float32),
        pltpu.VMEM((_VOCAB,), jnp.float32),
        pltpu.VMEM((_PER_W,), jnp.float32),
        pltpu.SemaphoreType.DMA,
    ],
)
def _sc_loss(emb_r8_hbm, idx_hbm, tgt_hbm, lse_hbm, loss_hbm,
             idx_all, tgt_all, seg_all, seg_v, lse_v, loss_all, sem):
    wid = lax.axis_index("s") * _NC + lax.axis_index("c")
    base_w = wid * _PER_W
    pltpu.sync_copy(idx_hbm.at[pl.ds(base_w, _PER_W)], idx_all)
    pltpu.sync_copy(tgt_hbm.at[pl.ds(base_w, _PER_W)], tgt_all)
    pltpu.sync_copy(lse_hbm, lse_v)

    for k in range(_PER_W // 16):
        sl = pl.ds(k * 16, 16)
        seg_all[sl] = idx_all[sl] * 8 + jnp.right_shift(tgt_all[sl], 7)

    def chunk(m, carry):
        off = pl.multiple_of(m * _SEG_C, _SEG_C)
        pltpu.async_copy(emb_r8_hbm.at[seg_all.at[pl.ds(off, _SEG_C)]],
                         seg_v, sem).wait()
        for j in range(_SEG_C // 16):
            sl = pl.ds(off + j * 16, 16)
            idx16 = idx_all[sl]
            lane16 = jnp.bitwise_and(tgt_all[sl], 127)
            row16 = lax.iota(jnp.int32, 16) + (j * 16)
            tl16 = plsc.load_gather(seg_v, [row16, lane16])
            lse16 = plsc.load_gather(lse_v, [idx16])
            loss_all[sl] = lse16 - tl16
        return carry

    lax.fori_loop(0, _NSEG_CHUNK, chunk, 0)
    pltpu.sync_copy(loss_all, loss_hbm.at[pl.ds(base_w, _PER_W)])


def kernel(indices, targets, embedding):
    emb_pad = jnp.pad(embedding, ((0, 0), (0, _VPAD - _VOCAB)))
    emb_r8 = emb_pad.reshape(_VOCAB * 8, 128)
    idx64 = jnp.pad(indices, ((0, 0), (0, _LP64 - _L))).reshape(_B * _LP64)
    idx_flat = indices.reshape(_N)
    tgt_flat = targets.reshape(_N)
    lse = _compute_lse(embedding)
    out_p = _sc_rows(emb_r8, idx64)
    loss = _sc_loss(emb_r8, idx_flat, tgt_flat, lse)
    return out_p[:, :_L, :_VOCAB], loss
